# baseline (speedup 1.0000x reference)
import functools

import jax
import jax.numpy as jnp
from jax import lax
from jax.experimental import pallas as pl
from jax.experimental.pallas import tpu as pltpu
from jax.experimental.pallas import tpu_sc as plsc

_D = 1024
_S = 4096
_B = 4
_NC = 2
_NS = 16
_NW = _NC * _NS
_ROWS_PER_W = _S // _NW
_CH = 32
_NBUF = 3
_NP = _ROWS_PER_W // _CH

_mesh = plsc.VectorSubcoreMesh(
    core_axis_name="c", subcore_axis_name="s", num_cores=_NC, num_subcores=_NS
)


@functools.partial(
    pl.kernel,
    mesh=_mesh,
    out_type=jax.ShapeDtypeStruct((_B, _S, _D), jnp.float32),
    scratch_types=[
        pltpu.VMEM((_NBUF, _CH, _D), jnp.float32),
        [pltpu.SemaphoreType.DMA] * _NBUF,
        [pltpu.SemaphoreType.DMA] * _NBUF,
    ],
)
def _pos_embed_sc(table_hbm, out_hbm, bufs, rsems, wsems):
    wid = lax.axis_index("s") * _NC + lax.axis_index("c")
    base = wid * _ROWS_PER_W

    def read(p):
        i = p % _NBUF
        return pltpu.async_copy(
            table_hbm.at[pl.ds(base + p * _CH, _CH)], bufs.at[i], rsems[i]
        )

    def write(p):
        i = p % _NBUF
        return [
            pltpu.async_copy(bufs.at[i], out_hbm.at[b, pl.ds(base + p * _CH, _CH)], wsems[i])
            for b in range(_B)
        ]

    reads = [read(p) for p in range(min(_NBUF, _NP))]
    writes = [None] * _NP
    for p in range(_NP):
        reads[p].wait()
        writes[p] = write(p)
        nxt = p + _NBUF
        if nxt < _NP:
            for c in writes[nxt - _NBUF]:
                c.wait()
            reads.append(read(nxt))
    for p in range(max(0, _NP - _NBUF), _NP):
        for c in writes[p]:
            c.wait()


def kernel(embeddings, table):
    del embeddings
    return _pos_embed_sc(table)
